# baseline (speedup 1.0000x reference)
import functools

import jax
import jax.numpy as jnp
from jax import lax
from jax.experimental import pallas as pl
from jax.experimental.pallas import tpu as pltpu
from jax.experimental.pallas import tpu_sc as plsc

_NUM_CORES = 2
_NUM_SUBCORES = 16
_NUM_WORKERS = _NUM_CORES * _NUM_SUBCORES

_CHUNK = 32
_NBUF = 3


def _make_copy_kernel(out_rows, dim, src_start):
    rows_per_w = out_rows // _NUM_WORKERS
    nchunks = rows_per_w // _CHUNK
    mesh = plsc.VectorSubcoreMesh(core_axis_name="c", subcore_axis_name="s")

    scratch = [
        pltpu.VMEM_SHARED((_NUM_SUBCORES, _NBUF, _CHUNK, dim), jnp.float32)
    ]
    scratch += [pltpu.SemaphoreType.DMA for _ in range(2 * _NBUF)]

    @functools.partial(
        pl.kernel,
        mesh=mesh,
        out_type=jax.ShapeDtypeStruct((out_rows, dim), jnp.float32),
        scratch_types=scratch,
    )
    def copy_kernel(table_hbm, out_hbm, *scratch_refs):
        shared = scratch_refs[0]
        in_sems = scratch_refs[1 : 1 + _NBUF]
        out_sems = scratch_refs[1 + _NBUF :]

        sid = lax.axis_index("s")
        wid = sid * _NUM_CORES + lax.axis_index("c")
        base = wid * rows_per_w

        def in_copy(c):
            b = c % _NBUF
            return pltpu.make_async_copy(
                table_hbm.at[pl.ds(src_start + base + c * _CHUNK, _CHUNK)],
                shared.at[sid, b],
                in_sems[b],
            )

        def out_copy(c):
            b = c % _NBUF
            return pltpu.make_async_copy(
                shared.at[sid, b],
                out_hbm.at[pl.ds(base + c * _CHUNK, _CHUNK)],
                out_sems[b],
            )

        in_copy(0).start()
        in_copy(1).start()
        for c in range(nchunks):
            in_copy(c).wait()
            out_copy(c).start()
            nxt = c + 2
            if nxt < nchunks:
                if nxt - _NBUF >= 0:
                    out_copy(nxt - _NBUF).wait()
                in_copy(nxt).start()
        for c in range(max(0, nchunks - _NBUF), nchunks):
            out_copy(c).wait()

    return copy_kernel


def kernel(input, weights):
    _, seq_len = input.shape
    num_emb, dim = weights.shape
    out_rows = 2 * seq_len
    src_start = num_emb // 2 - seq_len
    assert 0 <= src_start and src_start + out_rows <= num_emb
    assert out_rows % (_NUM_WORKERS * _CHUNK) == 0
    return _make_copy_kernel(out_rows, dim, src_start)(weights)
